# baseline (speedup 1.0000x reference)
import functools

import jax
import jax.numpy as jnp
from jax import lax
from jax.experimental import pallas as pl
from jax.experimental.pallas import tpu as pltpu
from jax.experimental.pallas import tpu_sc as plsc

NX = 512
NY = 512
C = 64
B = 2
P = 32000
S = NX * NY
OUTC = 2 * C
FLAT_OUT = B * OUTC * S

NC = 2
NS = 16
L = 16
NW = NC * NS
TS = (B * S) // NW
NPV = P // L
NOV = TS // L


def _idx_body(coords_ref, flat_ref):
    c = coords_ref[...]
    flat_ref[...] = (c[0:1] * S + c[1:2] + c[2:3] * NX + c[3:4])


def _flat_indices(coords_t):
    return pl.pallas_call(
        _idx_body,
        out_shape=jax.ShapeDtypeStruct((1, P), jnp.int32),
    )(coords_t)


RS = 64
NBS = NY // RS


def _stats_body(obsp_ref, g_ref, xs_ref):
    i = pl.program_id(1)
    y0 = i * RS
    rows = obsp_ref[0, pl.ds(y0, RS + 8), :]
    cols = []
    for di in range(3):
        for dj in range(3):
            sl = lax.slice(rows, (di, dj), (di + RS, dj + NX))
            cols.append(jnp.reshape(sl, (1, RS * NX)))
    x2 = jnp.concatenate(cols, axis=0)
    g = lax.dot_general(x2, x2, (((1,), (1,)), ((), ())),
                        preferred_element_type=jnp.float32)
    xs = jnp.sum(x2, axis=1, keepdims=True)

    @pl.when(jnp.logical_and(pl.program_id(0) == 0, i == 0))
    def _():
        g_ref[...] = jnp.zeros_like(g_ref)
        xs_ref[...] = jnp.zeros_like(xs_ref)

    g_ref[...] += g
    xs_ref[...] += xs


def _stats(obsp):
    return pl.pallas_call(
        _stats_body,
        grid=(B, NBS),
        in_specs=[pl.BlockSpec((1, NY + 8, NX + 2), lambda b, i: (b, 0, 0))],
        out_specs=[
            pl.BlockSpec((9, 9), lambda b, i: (0, 0)),
            pl.BlockSpec((9, 1), lambda b, i: (0, 0)),
        ],
        out_shape=[
            jax.ShapeDtypeStruct((9, 9), jnp.float32),
            jax.ShapeDtypeStruct((9, 1), jnp.float32),
        ],
    )(obsp)


RW = 16
NBW = NY // RW


def _write_body(obsp_ref, w9_ref, g_ref, xs_ref, gam_ref, bet_ref, out_ref):
    i = pl.program_id(1)
    y0 = i * RW

    w9 = w9_ref[...]
    n = jnp.float32(B * S)
    mean = lax.dot_general(w9, xs_ref[...], (((1,), (0,)), ((), ())),
                           preferred_element_type=jnp.float32) / n
    wg = lax.dot_general(w9, g_ref[...], (((1,), (0,)), ((), ())),
                         preferred_element_type=jnp.float32)
    ex2 = jnp.sum(wg * w9, axis=1, keepdims=True) / n
    var = ex2 - mean * mean
    inv = lax.rsqrt(var + 1e-3)
    scale = gam_ref[...][:, None] * inv
    shift = bet_ref[...][:, None] - mean * scale

    rows = obsp_ref[0, pl.ds(y0, RW + 8), :]
    cols = []
    for di in range(3):
        for dj in range(3):
            sl = lax.slice(rows, (di, dj), (di + RW, dj + NX))
            cols.append(jnp.reshape(sl, (1, RW * NX)))
    x2 = jnp.concatenate(cols, axis=0)
    conv = lax.dot_general(w9, x2, (((1,), (0,)), ((), ())),
                           preferred_element_type=jnp.float32)
    feat = jnp.maximum(conv * scale + shift, 0.0)
    out_ref[0, :C] = jnp.zeros((C, RW, NX), jnp.float32)
    out_ref[0, C:] = jnp.reshape(feat, (C, RW, NX))


def _write(obsp, w9, g, xs, gamma, beta):
    return pl.pallas_call(
        _write_body,
        grid=(B, NBW),
        in_specs=[
            pl.BlockSpec((1, NY + 8, NX + 2), lambda b, i: (b, 0, 0)),
            pl.BlockSpec((C, 9), lambda b, i: (0, 0)),
            pl.BlockSpec((9, 9), lambda b, i: (0, 0)),
            pl.BlockSpec((9, 1), lambda b, i: (0, 0)),
            pl.BlockSpec((C,), lambda b, i: (0,)),
            pl.BlockSpec((C,), lambda b, i: (0,)),
        ],
        out_specs=pl.BlockSpec((1, OUTC, RW, NX), lambda b, i: (b, 0, i, 0)),
        out_shape=jax.ShapeDtypeStruct((B, OUTC, NY, NX), jnp.float32),
    )(obsp, w9, g, xs, gamma, beta)


def _sc_body(canvas, flat_hbm, feat_hbm,
             idx_v, owner_v, pos_v, pid_v, rows_v, oidx_v, vals_v, obase_v,
             gsem, ssem):
    wid = lax.axis_index("s") * NC + lax.axis_index("c")
    base = wid * TS
    iota = lax.iota(jnp.int32, L)

    pltpu.sync_copy(flat_hbm, idx_v)

    def init_body(i, carry):
        owner_v[pl.ds(i * L, L)] = jnp.full((L,), -1, jnp.int32)
        return carry
    lax.fori_loop(0, NOV, init_body, 0, unroll=8)

    def own_body(i, carry):
        v = idx_v[pl.ds(i * L, L)]
        loc = v - base
        msk = jnp.logical_and(loc >= 0, loc < TS)
        locc = jnp.clip(loc, 0, TS - 1)
        plsc.store_scatter(owner_v, [locc], iota + i * L, mask=msk)
        return carry
    lax.fori_loop(0, NPV, own_body, 0, unroll=8)

    def cmp_body(i, n):
        o = owner_v[pl.ds(i * L, L)]
        msk = o >= 0
        plsc.store_compressed(pos_v.at[pl.ds(n, L)], iota + i * L, mask=msk)
        plsc.store_compressed(pid_v.at[pl.ds(n, L)], o, mask=msk)
        cnt = lax.reduce_max(plsc.all_reduce_population_count(msk), (0,))
        return n + cnt
    n = lax.fori_loop(0, NOV, cmp_body, 0)

    rem = lax.rem(n, L)

    @pl.when(rem > 0)
    def _():
        nb = n - rem
        pv = pos_v[pl.ds(nb, L)]
        dv = pid_v[pl.ds(nb, L)]
        neg = jnp.full((L,), jnp.iinfo(jnp.int32).min, jnp.int32)
        p0 = lax.reduce_max(jnp.where(iota == 0, pv, neg), (0,))
        d0 = lax.reduce_max(jnp.where(iota == 0, dv, neg), (0,))
        pos_v[pl.ds(nb, L)] = jnp.where(iota < rem, pv, p0)
        pid_v[pl.ds(nb, L)] = jnp.where(iota < rem, dv, d0)

    nv = lax.select(rem > 0, n // L + 1, n // L)
    nv8 = ((nv + 7) // 8) * 8

    @pl.when(jnp.logical_and(nv8 > nv, n > 0))
    def _():
        pv = pos_v[pl.ds(0, L)]
        dv = pid_v[pl.ds(0, L)]
        neg = jnp.full((L,), jnp.iinfo(jnp.int32).min, jnp.int32)
        p0 = lax.reduce_max(jnp.where(iota == 0, pv, neg), (0,))
        d0 = lax.reduce_max(jnp.where(iota == 0, dv, neg), (0,))

        def pad_body(j, carry):
            pos_v[pl.ds(j * L, L)] = jnp.broadcast_to(p0, (L,))
            pid_v[pl.ds(j * L, L)] = jnp.broadcast_to(d0, (L,))
            return carry
        lax.fori_loop(nv, nv8, pad_body, 0)

    CH = 8 * L

    def win_body(v, carry):
        st = v * CH
        cp = pltpu.async_copy(feat_hbm.at[pid_v.at[pl.ds(st, CH)]],
                              rows_v, gsem)

        def base_body(k, c2):
            posv = pos_v[pl.ds(st + k * L, L)]
            flatg = posv + base
            bidx = lax.shift_right_logical(flatg, 18)
            sidx = jnp.bitwise_and(flatg, S - 1)
            obase_v[pl.ds(k * L, L)] = lax.shift_left(bidx, 25) + sidx
            return c2
        lax.fori_loop(0, 8, base_body, 0, unroll=8)
        cp.wait()

        def ch_body(c, c2):
            def sub_body(k, c3):
                col = plsc.load_gather(
                    rows_v, [iota + k * L, jnp.full((L,), c, jnp.int32)])
                oidx_v[c, pl.ds(k * L, L)] = obase_v[pl.ds(k * L, L)] + c * S
                vals_v[c, pl.ds(k * L, L)] = col
                return c3
            lax.fori_loop(0, 8, sub_body, 0, unroll=8)
            return c2
        lax.fori_loop(0, C, ch_body, 0)

        def fire_body(c, c2):
            pltpu.async_copy(vals_v.at[c], canvas.at[oidx_v.at[c]], ssem)
            return c2
        lax.fori_loop(0, C, fire_body, 0)

        def drain_body(c, c2):
            pltpu.make_async_copy(vals_v.at[c], canvas.at[oidx_v.at[c]],
                                  ssem).wait()
            return c2
        lax.fori_loop(0, C, drain_body, 0)
        return carry
    lax.fori_loop(0, nv8 // 8, win_body, 0)


def _sc_scatter(canvas_ref, flat, feats):
    mesh = plsc.VectorSubcoreMesh(core_axis_name="c", subcore_axis_name="s",
                                  num_cores=NC, num_subcores=NS)
    k = pl.kernel(
        _sc_body,
        out_type=(),
        mesh=mesh,
        compiler_params=pltpu.CompilerParams(needs_layout_passes=False,
                                             use_tc_tiling_on_sc=False),
        scratch_types=[
            pltpu.VMEM((P,), jnp.int32),
            pltpu.VMEM((TS,), jnp.int32),
            pltpu.VMEM((TS,), jnp.int32),
            pltpu.VMEM((TS,), jnp.int32),
            pltpu.VMEM((8 * L, C), jnp.float32),
            pltpu.VMEM((C, 8 * L), jnp.int32),
            pltpu.VMEM((C, 8 * L), jnp.float32),
            pltpu.VMEM((8 * L,), jnp.int32),
            pltpu.SemaphoreType.DMA,
            pltpu.SemaphoreType.DMA,
        ],
    )
    return k(canvas_ref, flat, feats)


def kernel(pillar_features, voxel_coords, observations, conv_w, bn_gamma,
           bn_beta):
    obsp = jnp.pad(observations.reshape(B, NY, NX),
                   ((0, 0), (1, 7), (1, 1)))
    w9 = conv_w.reshape(C, 9)

    flat = _flat_indices(voxel_coords.T).reshape(P)
    g, xs = _stats(obsp)
    out = _write(obsp, w9, g, xs, bn_gamma, bn_beta)

    ref = jax.new_ref(jnp.reshape(out, (FLAT_OUT,)))
    _sc_scatter(ref, flat, pillar_features)
    return ref[...].reshape(B, OUTC, NY, NX)


# speedup vs baseline: 2.6591x; 2.6591x over previous
import functools

import jax
import jax.numpy as jnp
from jax import lax
from jax.experimental import pallas as pl
from jax.experimental.pallas import tpu as pltpu
from jax.experimental.pallas import tpu_sc as plsc

NX = 512
NY = 512
C = 64
B = 2
P = 32000
S = NX * NY
OUTC = 2 * C
FLAT_OUT = B * OUTC * S

NC = 2
NS = 16
L = 16
NW = NC * NS
TS = (B * S) // NW
NPV = P // L
NOV = TS // L


def _idx_body(coords_ref, flat_ref):
    c = coords_ref[...]
    flat_ref[...] = (c[0:1] * S + c[1:2] + c[2:3] * NX + c[3:4])


def _flat_indices(coords_t):
    return pl.pallas_call(
        _idx_body,
        out_shape=jax.ShapeDtypeStruct((1, P), jnp.int32),
    )(coords_t)


RS = 64
NBS = NY // RS


def _stats_body(obsp_ref, g_ref, xs_ref):
    i = pl.program_id(1)
    y0 = i * RS
    rows = obsp_ref[0, pl.ds(y0, RS + 8), :]
    cols = []
    for di in range(3):
        for dj in range(3):
            sl = lax.slice(rows, (di, dj), (di + RS, dj + NX))
            cols.append(jnp.reshape(sl, (1, RS * NX)))
    x2 = jnp.concatenate(cols, axis=0)
    g = lax.dot_general(x2, x2, (((1,), (1,)), ((), ())),
                        preferred_element_type=jnp.float32)
    xs = jnp.sum(x2, axis=1, keepdims=True)

    @pl.when(jnp.logical_and(pl.program_id(0) == 0, i == 0))
    def _():
        g_ref[...] = jnp.zeros_like(g_ref)
        xs_ref[...] = jnp.zeros_like(xs_ref)

    g_ref[...] += g
    xs_ref[...] += xs


def _stats(obsp):
    return pl.pallas_call(
        _stats_body,
        grid=(B, NBS),
        in_specs=[pl.BlockSpec((1, NY + 8, NX + 2), lambda b, i: (b, 0, 0))],
        out_specs=[
            pl.BlockSpec((9, 9), lambda b, i: (0, 0)),
            pl.BlockSpec((9, 1), lambda b, i: (0, 0)),
        ],
        out_shape=[
            jax.ShapeDtypeStruct((9, 9), jnp.float32),
            jax.ShapeDtypeStruct((9, 1), jnp.float32),
        ],
    )(obsp)


RW = 16
NBW = NY // RW


def _write_body(obsp_ref, w9_ref, g_ref, xs_ref, gam_ref, bet_ref, cv_ref,
                out_ref):
    i = pl.program_id(1)
    y0 = i * RW

    w9 = w9_ref[...]
    n = jnp.float32(B * S)
    mean = lax.dot_general(w9, xs_ref[...], (((1,), (0,)), ((), ())),
                           preferred_element_type=jnp.float32) / n
    wg = lax.dot_general(w9, g_ref[...], (((1,), (0,)), ((), ())),
                         preferred_element_type=jnp.float32)
    ex2 = jnp.sum(wg * w9, axis=1, keepdims=True) / n
    var = ex2 - mean * mean
    inv = lax.rsqrt(var + 1e-3)
    scale = gam_ref[...][:, None] * inv
    shift = bet_ref[...][:, None] - mean * scale

    rows = obsp_ref[0, pl.ds(y0, RW + 8), :]
    cols = []
    for di in range(3):
        for dj in range(3):
            sl = lax.slice(rows, (di, dj), (di + RW, dj + NX))
            cols.append(jnp.reshape(sl, (1, RW * NX)))
    x2 = jnp.concatenate(cols, axis=0)
    conv = lax.dot_general(w9, x2, (((1,), (0,)), ((), ())),
                           preferred_element_type=jnp.float32)
    feat = jnp.maximum(conv * scale + shift, 0.0)
    out_ref[0, :, :, :] = jnp.reshape(feat, (C, RW, NX))


def _write(obsp, w9, g, xs, gamma, beta, cv):
    return pl.pallas_call(
        _write_body,
        grid=(B, NBW),
        in_specs=[
            pl.BlockSpec((1, NY + 8, NX + 2), lambda b, i: (b, 0, 0)),
            pl.BlockSpec((C, 9), lambda b, i: (0, 0)),
            pl.BlockSpec((9, 9), lambda b, i: (0, 0)),
            pl.BlockSpec((9, 1), lambda b, i: (0, 0)),
            pl.BlockSpec((C,), lambda b, i: (0,)),
            pl.BlockSpec((C,), lambda b, i: (0,)),
            pl.BlockSpec(memory_space=pl.ANY),
        ],
        out_specs=pl.BlockSpec((1, C, RW, NX), lambda b, i: (b, 1, i, 0)),
        out_shape=jax.ShapeDtypeStruct((B, OUTC, NY, NX), jnp.float32),
        input_output_aliases={6: 0},
    )(obsp, w9, g, xs, gamma, beta, cv)


LPT = TS // NX
ROWS_PER_LINE = C * (NX // 128)
PPAD = 256 * 128
ICH = 32
NICH = PPAD // (128 * ICH)


def _sc_body(flat_hbm, feat_hbm, canvas,
             idx_c, owner_v, stag, rows_v, xl_v, pl_v, gix_v, six_v,
             gsem, ssem):
    wid = lax.axis_index("s") * NC + lax.axis_index("c")
    base = wid * TS
    iota = lax.iota(jnp.int32, L)

    def init_body(i, carry):
        owner_v[pl.ds(i * L, L)] = jnp.full((L,), -1, jnp.int32)
        return carry
    lax.fori_loop(0, NOV, init_body, 0, unroll=8)

    def zs_body(i, carry):
        stag[lax.shift_right_logical(i, 3),
             pl.ds((i % 8) * L, L)] = jnp.zeros((L,), jnp.float32)
        return carry
    lax.fori_loop(0, ROWS_PER_LINE * 8, zs_body, 0, unroll=8)

    def chunk_body(cc, carry):
        pltpu.sync_copy(flat_hbm.at[pl.ds(cc * ICH, ICH)], idx_c)

        def row_body(r, c2):
            def sub_body(j, c3):
                v = idx_c[r, pl.ds(j * L, L)]
                loc = v - base
                msk = jnp.logical_and(loc >= 0, loc < TS)
                locc = jnp.clip(loc, 0, TS - 1)
                pidv = (cc * ICH + r) * 128 + j * L + iota
                plsc.store_scatter(owner_v, [locc], pidv, mask=msk)
                return c3
            lax.fori_loop(0, 8, sub_body, 0, unroll=8)
            return c2
        lax.fori_loop(0, ICH, row_body, 0)
        return carry
    lax.fori_loop(0, NICH, chunk_body, 0)

    def line_body(l, carry):
        gl = wid * LPT + l
        rowbase = lax.shift_right_logical(gl, 9) * S \
            + jnp.bitwise_and(gl, NX - 1) * 4

        def cmp_body(i, nw):
            o = owner_v[pl.ds(l * NX + i * L, L)]
            msk = o >= 0
            plsc.store_compressed(xl_v.at[pl.ds(nw, L)], i * L + iota,
                                  mask=msk)
            plsc.store_compressed(pl_v.at[pl.ds(nw, L)], o, mask=msk)
            cnt = lax.reduce_max(plsc.all_reduce_population_count(msk), (0,))
            return nw + cnt
        nw = lax.fori_loop(0, NX // L, cmp_body, 0, unroll=4)

        def cc_body(cc, c2):
            s = cc * 128

            def gix_body(k, c3):
                pv = pl_v[pl.ds(s + k * L, L)]
                gix_v[pl.ds(k * L, L)] = jnp.clip(pv, 0, P - 1)
                return c3
            lax.fori_loop(0, 8, gix_body, 0, unroll=8)
            pltpu.async_copy(feat_hbm.at[gix_v], rows_v, gsem).wait()

            kk = jnp.minimum(nw - s, 128)
            kv = (kk + L - 1) // L

            def comp_body(k, c4):
                valid = (s + k * L + iota) < nw
                xv = jnp.clip(xl_v[pl.ds(s + k * L, L)], 0, NX - 1)
                r0 = lax.shift_right_logical(xv, 7)
                c0 = jnp.bitwise_and(xv, 127)

                def ch_body(c, c5):
                    col = plsc.load_gather(rows_v, [k * L + iota,
                                                    jnp.full((L,), c,
                                                             jnp.int32)])
                    plsc.store_scatter(stag, [c * 4 + r0, c0], col,
                                       mask=valid)
                    return c5
                lax.fori_loop(0, C, ch_body, 0, unroll=4)
                return c4
            lax.fori_loop(0, kv, comp_body, 0)
            return c2
        lax.fori_loop(0, (nw + 127) // 128, cc_body, 0)

        def six_body(t, c2):
            rv = t * L + iota
            six_v[lax.shift_right_logical(t, 3), pl.ds((t % 8) * L, L)] = (
                rowbase + lax.shift_right_logical(rv, 2) * (4 * NX)
                + jnp.bitwise_and(rv, 3))
            return c2
        lax.fori_loop(0, ROWS_PER_LINE // L, six_body, 0, unroll=8)

        pltpu.async_copy(stag.at[pl.ds(0, 128)], canvas.at[six_v.at[0]], ssem)
        pltpu.async_copy(stag.at[pl.ds(128, 128)], canvas.at[six_v.at[1]],
                         ssem)
        pltpu.make_async_copy(stag.at[pl.ds(0, 128)],
                              canvas.at[six_v.at[0]], ssem).wait()
        pltpu.make_async_copy(stag.at[pl.ds(128, 128)],
                              canvas.at[six_v.at[1]], ssem).wait()

        def rz_cc(cc, c2):
            s = cc * 128
            kk = jnp.minimum(nw - s, 128)
            kv = (kk + L - 1) // L

            def rz_k(k, c4):
                valid = (s + k * L + iota) < nw
                xv = jnp.clip(xl_v[pl.ds(s + k * L, L)], 0, NX - 1)
                r0 = lax.shift_right_logical(xv, 7)
                c0 = jnp.bitwise_and(xv, 127)
                zero = jnp.zeros((L,), jnp.float32)

                def rz_c(c, c5):
                    plsc.store_scatter(stag, [c * 4 + r0, c0], zero,
                                       mask=valid)
                    return c5
                lax.fori_loop(0, C, rz_c, 0, unroll=4)
                return c4
            lax.fori_loop(0, kv, rz_k, 0)
            return c2
        lax.fori_loop(0, (nw + 127) // 128, rz_cc, 0)
        return carry
    lax.fori_loop(0, LPT, line_body, 0)


def _sc_canvas(flat, feats):
    mesh = plsc.VectorSubcoreMesh(core_axis_name="c", subcore_axis_name="s",
                                  num_cores=NC, num_subcores=NS)
    k = pl.kernel(
        _sc_body,
        out_type=jax.ShapeDtypeStruct((FLAT_OUT // 128, 128), jnp.float32),
        mesh=mesh,
        compiler_params=pltpu.CompilerParams(needs_layout_passes=False,
                                             use_tc_tiling_on_sc=False),
        scratch_types=[
            pltpu.VMEM((ICH, 128), jnp.int32),
            pltpu.VMEM((TS,), jnp.int32),
            pltpu.VMEM((ROWS_PER_LINE, 128), jnp.float32),
            pltpu.VMEM((128, C), jnp.float32),
            pltpu.VMEM((NX,), jnp.int32),
            pltpu.VMEM((NX,), jnp.int32),
            pltpu.VMEM((128,), jnp.int32),
            pltpu.VMEM((2, 128), jnp.int32),
            pltpu.SemaphoreType.DMA,
            pltpu.SemaphoreType.DMA,
        ],
    )
    return k(flat, feats)


def kernel(pillar_features, voxel_coords, observations, conv_w, bn_gamma,
           bn_beta):
    obsp = jnp.pad(observations.reshape(B, NY, NX),
                   ((0, 0), (1, 7), (1, 1)))
    w9 = conv_w.reshape(C, 9)

    flat = jnp.pad(_flat_indices(voxel_coords.T).reshape(P // 128, 128),
                   ((0, (PPAD - P) // 128), (0, 0)),
                   constant_values=jnp.int32(2**30))
    cv = _sc_canvas(flat, pillar_features).reshape(B, OUTC, NY, NX)
    g, xs = _stats(obsp)
    return _write(obsp, w9, g, xs, bn_gamma, bn_beta, cv)


# speedup vs baseline: 3.3200x; 1.2486x over previous
import functools

import jax
import jax.numpy as jnp
from jax import lax
from jax.experimental import pallas as pl
from jax.experimental.pallas import tpu as pltpu
from jax.experimental.pallas import tpu_sc as plsc

NX = 512
NY = 512
C = 64
B = 2
P = 32000
S = NX * NY
OUTC = 2 * C
FLAT_OUT = B * OUTC * S

NC = 2
NS = 16
L = 16
NW = NC * NS
TS = (B * S) // NW
NPV = P // L
NOV = TS // L


def _idx_body(coords_ref, flat_ref):
    c = coords_ref[...]
    flat_ref[...] = (c[0:1] * S + c[1:2] + c[2:3] * NX + c[3:4])


def _flat_indices(coords_t):
    return pl.pallas_call(
        _idx_body,
        out_shape=jax.ShapeDtypeStruct((1, P), jnp.int32),
    )(coords_t)


RS = 64
NBS = NY // RS


def _stats_body(obsp_ref, g_ref, xs_ref):
    i = pl.program_id(1)
    y0 = i * RS
    rows = obsp_ref[0, pl.ds(y0, RS + 8), :]
    cols = []
    for di in range(3):
        for dj in range(3):
            sl = lax.slice(rows, (di, dj), (di + RS, dj + NX))
            cols.append(jnp.reshape(sl, (1, RS * NX)))
    x2 = jnp.concatenate(cols, axis=0)
    g = lax.dot_general(x2, x2, (((1,), (1,)), ((), ())),
                        preferred_element_type=jnp.float32)
    xs = jnp.sum(x2, axis=1, keepdims=True)

    @pl.when(jnp.logical_and(pl.program_id(0) == 0, i == 0))
    def _():
        g_ref[...] = jnp.zeros_like(g_ref)
        xs_ref[...] = jnp.zeros_like(xs_ref)

    g_ref[...] += g
    xs_ref[...] += xs


def _stats(obsp):
    return pl.pallas_call(
        _stats_body,
        grid=(B, NBS),
        in_specs=[pl.BlockSpec((1, NY + 8, NX + 2), lambda b, i: (b, 0, 0))],
        out_specs=[
            pl.BlockSpec((9, 9), lambda b, i: (0, 0)),
            pl.BlockSpec((9, 1), lambda b, i: (0, 0)),
        ],
        out_shape=[
            jax.ShapeDtypeStruct((9, 9), jnp.float32),
            jax.ShapeDtypeStruct((9, 1), jnp.float32),
        ],
    )(obsp)


RW = 16
NBW = NY // RW


def _write_body(obsp_ref, w9_ref, g_ref, xs_ref, gam_ref, bet_ref, cv_ref,
                out_ref):
    i = pl.program_id(1)
    y0 = i * RW

    w9 = w9_ref[...]
    n = jnp.float32(B * S)
    mean = lax.dot_general(w9, xs_ref[...], (((1,), (0,)), ((), ())),
                           preferred_element_type=jnp.float32) / n
    wg = lax.dot_general(w9, g_ref[...], (((1,), (0,)), ((), ())),
                         preferred_element_type=jnp.float32)
    ex2 = jnp.sum(wg * w9, axis=1, keepdims=True) / n
    var = ex2 - mean * mean
    inv = lax.rsqrt(var + 1e-3)
    scale = gam_ref[...][:, None] * inv
    shift = bet_ref[...][:, None] - mean * scale

    rows = obsp_ref[0, pl.ds(y0, RW + 8), :]
    cols = []
    for di in range(3):
        for dj in range(3):
            sl = lax.slice(rows, (di, dj), (di + RW, dj + NX))
            cols.append(jnp.reshape(sl, (1, RW * NX)))
    x2 = jnp.concatenate(cols, axis=0)
    conv = lax.dot_general(w9, x2, (((1,), (0,)), ((), ())),
                           preferred_element_type=jnp.float32)
    feat = jnp.maximum(conv * scale + shift, 0.0)
    out_ref[0, :, :, :] = jnp.reshape(feat, (C, RW, NX))


def _write(obsp, w9, g, xs, gamma, beta, cv):
    return pl.pallas_call(
        _write_body,
        grid=(B, NBW),
        in_specs=[
            pl.BlockSpec((1, NY + 8, NX + 2), lambda b, i: (b, 0, 0)),
            pl.BlockSpec((C, 9), lambda b, i: (0, 0)),
            pl.BlockSpec((9, 9), lambda b, i: (0, 0)),
            pl.BlockSpec((9, 1), lambda b, i: (0, 0)),
            pl.BlockSpec((C,), lambda b, i: (0,)),
            pl.BlockSpec((C,), lambda b, i: (0,)),
            pl.BlockSpec(memory_space=pl.ANY),
        ],
        out_specs=pl.BlockSpec((1, C, RW, NX), lambda b, i: (b, 1, i, 0)),
        out_shape=jax.ShapeDtypeStruct((B, OUTC, NY, NX), jnp.float32),
        input_output_aliases={6: 0},
    )(obsp, w9, g, xs, gamma, beta, cv)


LPT = TS // NX
ROWS_PER_LINE = C * (NX // 128)
PPAD = 256 * 128
ICH = 32
NICH = PPAD // (128 * ICH)


def _sc_body(flat_hbm, feat_hbm, canvas,
             idx_c, owner_v, stag, rows_v, xl_v, pl_v, gix_v,
             gsem, ssem):
    wid = lax.axis_index("s") * NC + lax.axis_index("c")
    base = wid * TS
    iota = lax.iota(jnp.int32, L)

    def init_body(i, carry):
        owner_v[pl.ds(i * L, L)] = jnp.full((L,), -1, jnp.int32)
        return carry
    lax.fori_loop(0, NOV, init_body, 0, unroll=8)

    def zs_body(i, carry):
        stag[lax.shift_right_logical(i, 3),
             pl.ds((i % 8) * L, L)] = jnp.zeros((L,), jnp.float32)
        return carry
    lax.fori_loop(0, ROWS_PER_LINE * 8, zs_body, 0, unroll=8)

    def chunk_body(cc, carry):
        pltpu.sync_copy(flat_hbm.at[pl.ds(cc * ICH, ICH)], idx_c)

        def row_body(r, c2):
            def sub_body(j, c3):
                v = idx_c[r, pl.ds(j * L, L)]
                loc = v - base
                msk = jnp.logical_and(loc >= 0, loc < TS)
                locc = jnp.clip(loc, 0, TS - 1)
                pidv = (cc * ICH + r) * 128 + j * L + iota
                plsc.store_scatter(owner_v, [locc], pidv, mask=msk)
                return c3
            lax.fori_loop(0, 8, sub_body, 0, unroll=8)
            return c2
        lax.fori_loop(0, ICH, row_body, 0)
        return carry
    lax.fori_loop(0, NICH, chunk_body, 0)

    def pair_body(p, carry):
        gp = wid * (LPT // 2) + p
        rowbase = lax.shift_right_logical(gp, 8) * S \
            + jnp.bitwise_and(gp, 255) * 8

        def cmp_body(i, nw):
            o = owner_v[pl.ds(p * 2 * NX + i * L, L)]
            msk = o >= 0
            plsc.store_compressed(xl_v.at[pl.ds(nw, L)], i * L + iota,
                                  mask=msk)
            plsc.store_compressed(pl_v.at[pl.ds(nw, L)], o, mask=msk)
            cnt = lax.reduce_max(plsc.all_reduce_population_count(msk), (0,))
            return nw + cnt
        nw = lax.fori_loop(0, 2 * NX // L, cmp_body, 0, unroll=4)

        for h in range(2):
            def cc_body(cc, c2):
                s = cc * 128

                def gix_body(k, c3):
                    pv = pl_v[pl.ds(s + k * L, L)]
                    gix_v[pl.ds(k * L, L)] = jnp.clip(pv, 0, P - 1)
                    return c3
                lax.fori_loop(0, 8, gix_body, 0, unroll=8)
                pltpu.async_copy(feat_hbm.at[gix_v], rows_v, gsem).wait()

                kk = jnp.minimum(nw - s, 128)
                kv = (kk + L - 1) // L

                def comp_body(k, c4):
                    valid = (s + k * L + iota) < nw
                    xv = jnp.clip(xl_v[pl.ds(s + k * L, L)], 0, 2 * NX - 1)
                    r0 = lax.shift_right_logical(xv, 7)
                    c0 = jnp.bitwise_and(xv, 127)

                    def ch_body(c, c5):
                        col = plsc.load_gather(
                            rows_v, [k * L + iota,
                                     jnp.full((L,), h * (C // 2), jnp.int32)
                                     + c])
                        plsc.store_scatter(stag, [c * 8 + r0, c0], col,
                                           mask=valid)
                        return c5
                    lax.fori_loop(0, C // 2, ch_body, 0, unroll=4)
                    return c4
                lax.fori_loop(0, kv, comp_body, 0)
                return c2
            lax.fori_loop(0, (nw + 127) // 128, cc_body, 0)

            for c in range(C // 2):
                off = pl.multiple_of(
                    rowbase + (h * (C // 2) + c) * (4 * NX // 2) * 2, 8)
                pltpu.async_copy(stag.at[pl.ds(c * 8, 8)],
                                 canvas.at[pl.ds(off, 8)], ssem)
            for c in range(C // 2):
                off = pl.multiple_of(
                    rowbase + (h * (C // 2) + c) * (4 * NX // 2) * 2, 8)
                pltpu.make_async_copy(stag.at[pl.ds(c * 8, 8)],
                                      canvas.at[pl.ds(off, 8)], ssem).wait()

            def rz_cc(cc, c2):
                s = cc * 128
                kk = jnp.minimum(nw - s, 128)
                kv = (kk + L - 1) // L

                def rz_k(k, c4):
                    valid = (s + k * L + iota) < nw
                    xv = jnp.clip(xl_v[pl.ds(s + k * L, L)], 0, 2 * NX - 1)
                    r0 = lax.shift_right_logical(xv, 7)
                    c0 = jnp.bitwise_and(xv, 127)
                    zero = jnp.zeros((L,), jnp.float32)

                    def rz_c(c, c5):
                        plsc.store_scatter(stag, [c * 8 + r0, c0], zero,
                                           mask=valid)
                        return c5
                    lax.fori_loop(0, C // 2, rz_c, 0, unroll=4)
                    return c4
                lax.fori_loop(0, kv, rz_k, 0)
                return c2
            lax.fori_loop(0, (nw + 127) // 128, rz_cc, 0)
        return carry
    lax.fori_loop(0, LPT // 2, pair_body, 0)


def _sc_canvas(flat, feats):
    mesh = plsc.VectorSubcoreMesh(core_axis_name="c", subcore_axis_name="s",
                                  num_cores=NC, num_subcores=NS)
    k = pl.kernel(
        _sc_body,
        out_type=jax.ShapeDtypeStruct((FLAT_OUT // 128, 128), jnp.float32),
        mesh=mesh,
        compiler_params=pltpu.CompilerParams(needs_layout_passes=False,
                                             use_tc_tiling_on_sc=False),
        scratch_types=[
            pltpu.VMEM((ICH, 128), jnp.int32),
            pltpu.VMEM((TS,), jnp.int32),
            pltpu.VMEM((ROWS_PER_LINE, 128), jnp.float32),
            pltpu.VMEM((128, C), jnp.float32),
            pltpu.VMEM((2 * NX,), jnp.int32),
            pltpu.VMEM((2 * NX,), jnp.int32),
            pltpu.VMEM((128,), jnp.int32),
            pltpu.SemaphoreType.DMA,
            pltpu.SemaphoreType.DMA,
        ],
    )
    return k(flat, feats)


def kernel(pillar_features, voxel_coords, observations, conv_w, bn_gamma,
           bn_beta):
    obsp = jnp.pad(observations.reshape(B, NY, NX),
                   ((0, 0), (1, 7), (1, 1)))
    w9 = conv_w.reshape(C, 9)

    flat = jnp.pad(_flat_indices(voxel_coords.T).reshape(P // 128, 128),
                   ((0, (PPAD - P) // 128), (0, 0)),
                   constant_values=jnp.int32(2**30))
    cv = _sc_canvas(flat, pillar_features).reshape(B, OUTC, NY, NX)
    g, xs = _stats(obsp)
    return _write(obsp, w9, g, xs, bn_gamma, bn_beta, cv)


# speedup vs baseline: 4.4830x; 1.3503x over previous
import functools

import jax
import jax.numpy as jnp
from jax import lax
from jax.experimental import pallas as pl
from jax.experimental.pallas import tpu as pltpu
from jax.experimental.pallas import tpu_sc as plsc

NX = 512
NY = 512
C = 64
B = 2
P = 32000
S = NX * NY
OUTC = 2 * C
FLAT_OUT = B * OUTC * S

NC = 2
NS = 16
L = 16
NW = NC * NS
TS = (B * S) // NW
NPV = P // L
NOV = TS // L


def _idx_body(coords_ref, flat_ref):
    c = coords_ref[...]
    flat_ref[...] = (c[0:1] * S + c[1:2] + c[2:3] * NX + c[3:4])


def _flat_indices(coords_t):
    return pl.pallas_call(
        _idx_body,
        out_shape=jax.ShapeDtypeStruct((1, P), jnp.int32),
    )(coords_t)


RS = 64
NBS = NY // RS


def _stats_body(obsp_ref, g_ref, xs_ref):
    i = pl.program_id(1)
    y0 = i * RS
    rows = obsp_ref[0, pl.ds(y0, RS + 8), :]
    cols = []
    for di in range(3):
        for dj in range(3):
            sl = lax.slice(rows, (di, dj), (di + RS, dj + NX))
            cols.append(jnp.reshape(sl, (1, RS * NX)))
    x2 = jnp.concatenate(cols, axis=0)
    g = lax.dot_general(x2, x2, (((1,), (1,)), ((), ())),
                        preferred_element_type=jnp.float32)
    xs = jnp.sum(x2, axis=1, keepdims=True)

    @pl.when(jnp.logical_and(pl.program_id(0) == 0, i == 0))
    def _():
        g_ref[...] = jnp.zeros_like(g_ref)
        xs_ref[...] = jnp.zeros_like(xs_ref)

    g_ref[...] += g
    xs_ref[...] += xs


def _stats(obsp):
    return pl.pallas_call(
        _stats_body,
        grid=(B, NBS),
        in_specs=[pl.BlockSpec((1, NY + 8, NX + 2), lambda b, i: (b, 0, 0))],
        out_specs=[
            pl.BlockSpec((9, 9), lambda b, i: (0, 0)),
            pl.BlockSpec((9, 1), lambda b, i: (0, 0)),
        ],
        out_shape=[
            jax.ShapeDtypeStruct((9, 9), jnp.float32),
            jax.ShapeDtypeStruct((9, 1), jnp.float32),
        ],
    )(obsp)


RW = 16
NBW = NY // RW


def _write_body(obsp_ref, w9_ref, g_ref, xs_ref, gam_ref, bet_ref, cv_ref,
                out_ref):
    i = pl.program_id(1)
    y0 = i * RW

    w9 = w9_ref[...]
    n = jnp.float32(B * S)
    mean = lax.dot_general(w9, xs_ref[...], (((1,), (0,)), ((), ())),
                           preferred_element_type=jnp.float32) / n
    wg = lax.dot_general(w9, g_ref[...], (((1,), (0,)), ((), ())),
                         preferred_element_type=jnp.float32)
    ex2 = jnp.sum(wg * w9, axis=1, keepdims=True) / n
    var = ex2 - mean * mean
    inv = lax.rsqrt(var + 1e-3)
    scale = gam_ref[...][:, None] * inv
    shift = bet_ref[...][:, None] - mean * scale

    rows = obsp_ref[0, pl.ds(y0, RW + 8), :]
    cols = []
    for di in range(3):
        for dj in range(3):
            sl = lax.slice(rows, (di, dj), (di + RW, dj + NX))
            cols.append(jnp.reshape(sl, (1, RW * NX)))
    x2 = jnp.concatenate(cols, axis=0)
    conv = lax.dot_general(w9, x2, (((1,), (0,)), ((), ())),
                           preferred_element_type=jnp.float32)
    feat = jnp.maximum(conv * scale + shift, 0.0)
    out_ref[0, :, :, :] = jnp.reshape(feat, (C, RW, NX))


def _write(obsp, w9, g, xs, gamma, beta, cv):
    return pl.pallas_call(
        _write_body,
        grid=(B, NBW),
        in_specs=[
            pl.BlockSpec((1, NY + 8, NX + 2), lambda b, i: (b, 0, 0)),
            pl.BlockSpec((C, 9), lambda b, i: (0, 0)),
            pl.BlockSpec((9, 9), lambda b, i: (0, 0)),
            pl.BlockSpec((9, 1), lambda b, i: (0, 0)),
            pl.BlockSpec((C,), lambda b, i: (0,)),
            pl.BlockSpec((C,), lambda b, i: (0,)),
            pl.BlockSpec(memory_space=pl.ANY),
        ],
        out_specs=pl.BlockSpec((1, C, RW, NX), lambda b, i: (b, 1, i, 0)),
        out_shape=jax.ShapeDtypeStruct((B, OUTC, NY, NX), jnp.float32),
        input_output_aliases={6: 0},
    )(obsp, w9, g, xs, gamma, beta, cv)


LPT = TS // NX
ROWS_PER_LINE = C * (NX // 128)
PPAD = 256 * 128
ICH = 32
NICH = PPAD // (128 * ICH)


def _sc_body(flat_hbm, feat_hbm, canvas,
             idx_c, owner_v, stag, rows_v, xl_v, pl_v, gix_v,
             gsem, ssem):
    wid = lax.axis_index("s") * NC + lax.axis_index("c")
    base = wid * TS
    iota = lax.iota(jnp.int32, L)

    def init_body(i, carry):
        owner_v[pl.ds(i * L, L)] = jnp.full((L,), -1, jnp.int32)
        return carry
    lax.fori_loop(0, NOV, init_body, 0, unroll=8)

    def zs_body(i, carry):
        stag[lax.shift_right_logical(i, 3),
             pl.ds((i % 8) * L, L)] = jnp.zeros((L,), jnp.float32)
        return carry
    lax.fori_loop(0, ROWS_PER_LINE * 8, zs_body, 0, unroll=8)

    def chunk_body(cc, carry):
        pltpu.sync_copy(flat_hbm.at[pl.ds(cc * ICH, ICH)], idx_c)

        def row_body(r, c2):
            def sub_body(j, c3):
                v = idx_c[r, pl.ds(j * L, L)]
                loc = v - base
                msk = jnp.logical_and(loc >= 0, loc < TS)
                locc = jnp.clip(loc, 0, TS - 1)
                pidv = (cc * ICH + r) * 128 + j * L + iota
                plsc.store_scatter(owner_v, [locc], pidv, mask=msk)
                return c3
            lax.fori_loop(0, 8, sub_body, 0, unroll=8)
            return c2
        lax.fori_loop(0, ICH, row_body, 0)
        return carry
    lax.fori_loop(0, NICH, chunk_body, 0)

    def pair_body(p, carry):
        gp = wid * (LPT // 2) + p
        rowbase = lax.shift_right_logical(gp, 8) * S \
            + jnp.bitwise_and(gp, 255) * 8

        def cmp_body(i, nw):
            o = owner_v[pl.ds(p * 2 * NX + i * L, L)]
            msk = o >= 0
            plsc.store_compressed(xl_v.at[pl.ds(nw, L)], i * L + iota,
                                  mask=msk)
            plsc.store_compressed(pl_v.at[pl.ds(nw, L)], o, mask=msk)
            cnt = lax.reduce_max(plsc.all_reduce_population_count(msk), (0,))
            return nw + cnt
        nw = lax.fori_loop(0, 2 * NX // L, cmp_body, 0, unroll=4)

        for h in range(2):
            def cc_body(cc, c2):
                s = cc * 128

                def gix_body(k, c3):
                    pv = pl_v[pl.ds(s + k * L, L)]
                    gix_v[pl.ds(k * L, L)] = jnp.clip(pv, 0, P - 1)
                    return c3
                if h == 0:
                    lax.fori_loop(0, 8, gix_body, 0, unroll=8)
                    pltpu.async_copy(feat_hbm.at[gix_v], rows_v, gsem).wait()
                else:
                    @pl.when(nw > 128)
                    def _():
                        lax.fori_loop(0, 8, gix_body, 0, unroll=8)
                        pltpu.async_copy(feat_hbm.at[gix_v], rows_v,
                                         gsem).wait()

                kk = jnp.minimum(nw - s, 128)
                kv = (kk + L - 1) // L

                def comp_body(k, c4):
                    valid = (s + k * L + iota) < nw
                    xv = jnp.clip(xl_v[pl.ds(s + k * L, L)], 0, 2 * NX - 1)
                    r0 = lax.shift_right_logical(xv, 7)
                    c0 = jnp.bitwise_and(xv, 127)

                    def ch_body(c, c5):
                        col = plsc.load_gather(
                            rows_v, [k * L + iota,
                                     jnp.full((L,), h * (C // 2), jnp.int32)
                                     + c])
                        plsc.store_scatter(stag, [c * 8 + r0, c0], col,
                                           mask=valid)
                        return c5
                    lax.fori_loop(0, C // 2, ch_body, 0, unroll=4)
                    return c4
                lax.fori_loop(0, kv, comp_body, 0)
                return c2
            lax.fori_loop(0, (nw + 127) // 128, cc_body, 0)

            for c in range(C // 2):
                off = pl.multiple_of(
                    rowbase + (h * (C // 2) + c) * (4 * NX // 2) * 2, 8)
                pltpu.async_copy(stag.at[pl.ds(c * 8, 8)],
                                 canvas.at[pl.ds(off, 8)], ssem)
            for c in range(C // 2):
                off = pl.multiple_of(
                    rowbase + (h * (C // 2) + c) * (4 * NX // 2) * 2, 8)
                pltpu.make_async_copy(stag.at[pl.ds(c * 8, 8)],
                                      canvas.at[pl.ds(off, 8)], ssem).wait()

            def rz_cc(cc, c2):
                s = cc * 128
                kk = jnp.minimum(nw - s, 128)
                kv = (kk + L - 1) // L

                def rz_k(k, c4):
                    valid = (s + k * L + iota) < nw
                    xv = jnp.clip(xl_v[pl.ds(s + k * L, L)], 0, 2 * NX - 1)
                    r0 = lax.shift_right_logical(xv, 7)
                    c0 = jnp.bitwise_and(xv, 127)
                    zero = jnp.zeros((L,), jnp.float32)

                    def rz_c(c, c5):
                        plsc.store_scatter(stag, [c * 8 + r0, c0], zero,
                                           mask=valid)
                        return c5
                    lax.fori_loop(0, C // 2, rz_c, 0, unroll=4)
                    return c4
                lax.fori_loop(0, kv, rz_k, 0)
                return c2
            lax.fori_loop(0, (nw + 127) // 128, rz_cc, 0)
        return carry
    lax.fori_loop(0, LPT // 2, pair_body, 0)


def _sc_canvas(flat, feats):
    mesh = plsc.VectorSubcoreMesh(core_axis_name="c", subcore_axis_name="s",
                                  num_cores=NC, num_subcores=NS)
    k = pl.kernel(
        _sc_body,
        out_type=jax.ShapeDtypeStruct((FLAT_OUT // 128, 128), jnp.float32),
        mesh=mesh,
        compiler_params=pltpu.CompilerParams(needs_layout_passes=False,
                                             use_tc_tiling_on_sc=False),
        scratch_types=[
            pltpu.VMEM((ICH, 128), jnp.int32),
            pltpu.VMEM((TS,), jnp.int32),
            pltpu.VMEM((ROWS_PER_LINE, 128), jnp.float32),
            pltpu.VMEM((128, C), jnp.float32),
            pltpu.VMEM((2 * NX,), jnp.int32),
            pltpu.VMEM((2 * NX,), jnp.int32),
            pltpu.VMEM((128,), jnp.int32),
            pltpu.SemaphoreType.DMA,
            pltpu.SemaphoreType.DMA,
        ],
    )
    return k(flat, feats)


def kernel(pillar_features, voxel_coords, observations, conv_w, bn_gamma,
           bn_beta):
    obsp = jnp.pad(observations.reshape(B, NY, NX),
                   ((0, 0), (1, 7), (1, 1)))
    w9 = conv_w.reshape(C, 9)

    flat = jnp.pad(_flat_indices(voxel_coords.T).reshape(P // 128, 128),
                   ((0, (PPAD - P) // 128), (0, 0)),
                   constant_values=jnp.int32(2**30))
    cv = _sc_canvas(flat, pillar_features).reshape(B, OUTC, NY, NX)
    g, xs = _stats(obsp)
    return _write(obsp, w9, g, xs, bn_gamma, bn_beta, cv)
